# baseline (speedup 1.0000x reference)
import functools

import jax
import jax.numpy as jnp
from jax import lax
from jax.experimental import pallas as pl
from jax.experimental.pallas import tpu as pltpu
from jax.experimental.pallas import tpu_sc as plsc

D_IN = 768
NUM_LATENTS = 12288
K = 32
N_TOKENS = 4096

TOK_BLK = 256
LAT_TILE = 1536

NW = 32
B_PER_W = N_TOKENS // NW
LANES = 16


def _encode_topk_kernel(x_ref, wenc_ref, benc_ref, bdec_ref,
                        vals_ref, idxs_ref, s0_ref, s1_ref):
    B = x_ref.shape[0]
    L = wenc_ref.shape[0]
    T = LAT_TILE
    sae_in = x_ref[...] - bdec_ref[...]

    iota_t = lax.broadcasted_iota(jnp.int32, (B, T), 1)
    iota2k = lax.broadcasted_iota(jnp.int32, (B, 2 * K), 1)
    kiota = lax.broadcasted_iota(jnp.int32, (B, K), 1)
    big = jnp.int32(2 ** 30)

    run_v = jnp.full((B, K), -1.0, jnp.float32)
    run_i = jnp.full((B, K), big, jnp.int32)
    scratches = (s0_ref, s1_ref)

    for t in range(L // T):
        s = scratches[t % 2]
        w = wenc_ref[t * T:(t + 1) * T, :]
        a = lax.dot_general(sae_in, w, (((1,), (1,)), ((), ())),
                            preferred_element_type=jnp.float32)
        b = benc_ref[0, t * T:(t + 1) * T][None, :]
        s[...] = jnp.maximum(a + b, 0.0)

        def tbody(k, carry, s=s, t=t):
            tv, ti = carry
            av = s[...]
            m = jnp.max(av, axis=1, keepdims=True)
            idx = jnp.min(jnp.where(av == m, iota_t, big),
                          axis=1, keepdims=True)
            s[...] = jnp.where(iota_t == idx, -1.0, av)
            tv = jnp.where(kiota == k, m, tv)
            ti = jnp.where(kiota == k, idx + t * T, ti)
            return tv, ti

        tv, ti = lax.fori_loop(
            0, K, tbody,
            (jnp.zeros((B, K), jnp.float32), jnp.zeros((B, K), jnp.int32)))

        if t == 0:
            run_v, run_i = tv, ti
        else:
            cv = jnp.concatenate([run_v, tv], axis=1)
            ci = jnp.concatenate([run_i, ti], axis=1)

            def mbody(k, carry):
                rv, ri, c = carry
                m = jnp.max(c, axis=1, keepdims=True)
                pos = jnp.min(jnp.where(c == m, iota2k, big),
                              axis=1, keepdims=True)
                ii = jnp.min(jnp.where(iota2k == pos, ci, big),
                             axis=1, keepdims=True)
                c = jnp.where(iota2k == pos, -2.0, c)
                rv = jnp.where(kiota == k, m, rv)
                ri = jnp.where(kiota == k, ii, ri)
                return rv, ri, c

            run_v, run_i, _ = lax.fori_loop(
                0, K, mbody,
                (jnp.zeros((B, K), jnp.float32),
                 jnp.zeros((B, K), jnp.int32), cv))

    vals_ref[...] = run_v
    idxs_ref[...] = run_i


def _loss_kernel(x_ref, out_ref, l2_ref, tv_ref):
    xv = x_ref[...]
    ov = out_ref[...]
    e = ov - xv
    l2_ref[0, 0] = jnp.sum(e * e)
    mu = jnp.mean(xv, axis=0, keepdims=True)
    d = xv - mu
    tv_ref[0, 0] = jnp.sum(d * d)


def _decode_kernel(wdec_hbm, idx_hbm, vals_hbm, bdec_hbm, out_hbm,
                   idx_v, vals_v, rows_v, bcast_v, bdec_v, orow_v, sem0, sem1):
    wid = lax.axis_index("s") * 2 + lax.axis_index("c")
    base = wid * B_PER_W
    pltpu.sync_copy(idx_hbm.at[pl.ds(base, B_PER_W)], idx_v)
    pltpu.sync_copy(vals_hbm.at[pl.ds(base, B_PER_W)], vals_v)
    pltpu.sync_copy(bdec_hbm, bdec_v)

    lane = lax.iota(jnp.int32, LANES)
    sems = (sem0, sem1)

    pltpu.async_copy(wdec_hbm.at[idx_v.at[0]], rows_v.at[0], sems[0])

    def pair_body(p, carry):
        for b in range(2):
            t = 2 * p + b
            nxt = 1 - b

            @pl.when(t + 1 < B_PER_W)
            def _():
                pltpu.async_copy(wdec_hbm.at[idx_v.at[t + 1]],
                                 rows_v.at[nxt], sems[nxt])

            vrow_a = vals_v[t, pl.ds(0, LANES)]
            vrow_b = vals_v[t, pl.ds(LANES, LANES)]
            for k in range(K):
                src = vrow_a if k < LANES else vrow_b
                bval = jnp.sum(jnp.where(lane == (k % LANES), src, 0.0))
                bcast_v[k, :] = jnp.full((LANES,), bval)

            pltpu.make_async_copy(wdec_hbm.at[idx_v.at[t]],
                                  rows_v.at[b], sems[b]).wait()
            rbuf = rows_v.at[b]
            for g in range(3):
                def kbody(k, acc):
                    bval = bcast_v[k, pl.ds(0, LANES)]
                    return tuple(
                        acc[c] + bval * rbuf[k, pl.ds((g * 16 + c) * LANES, LANES)]
                        for c in range(16))
                init = tuple(bdec_v[pl.ds((g * 16 + c) * LANES, LANES)]
                             for c in range(16))
                acc = lax.fori_loop(0, K, kbody, init)
                for c in range(16):
                    orow_v[pl.ds((g * 16 + c) * LANES, LANES)] = acc[c]
            pltpu.sync_copy(orow_v, out_hbm.at[base + t])
        return carry

    lax.fori_loop(0, B_PER_W // 2, pair_body, 0)


def _encode_topk(x, W_enc, b_enc, b_dec):
    grid = (N_TOKENS // TOK_BLK,)
    return pl.pallas_call(
        _encode_topk_kernel,
        grid=grid,
        in_specs=[
            pl.BlockSpec((TOK_BLK, D_IN), lambda i: (i, 0)),
            pl.BlockSpec((NUM_LATENTS, D_IN), lambda i: (0, 0)),
            pl.BlockSpec((1, NUM_LATENTS), lambda i: (0, 0)),
            pl.BlockSpec((1, D_IN), lambda i: (0, 0)),
        ],
        out_specs=[
            pl.BlockSpec((TOK_BLK, K), lambda i: (i, 0)),
            pl.BlockSpec((TOK_BLK, K), lambda i: (i, 0)),
        ],
        out_shape=[
            jax.ShapeDtypeStruct((N_TOKENS, K), jnp.float32),
            jax.ShapeDtypeStruct((N_TOKENS, K), jnp.int32),
        ],
        scratch_shapes=[pltpu.VMEM((TOK_BLK, LAT_TILE), jnp.float32),
                        pltpu.VMEM((TOK_BLK, LAT_TILE), jnp.float32)],
        compiler_params=pltpu.CompilerParams(
            vmem_limit_bytes=100 * 1024 * 1024),
    )(x, W_enc, b_enc.reshape(1, NUM_LATENTS), b_dec.reshape(1, D_IN))


def _decode(W_dec, top_idx, top_vals, b_dec):
    mesh = plsc.VectorSubcoreMesh(core_axis_name="c", subcore_axis_name="s")
    fn = functools.partial(
        pl.kernel,
        mesh=mesh,
        out_type=jax.ShapeDtypeStruct((N_TOKENS, D_IN), jnp.float32),
        scratch_types=[
            pltpu.VMEM((B_PER_W, K), jnp.int32),
            pltpu.VMEM((B_PER_W, K), jnp.float32),
            pltpu.VMEM((2, K, D_IN), jnp.float32),
            pltpu.VMEM((K, LANES), jnp.float32),
            pltpu.VMEM((D_IN,), jnp.float32),
            pltpu.VMEM((D_IN,), jnp.float32),
            pltpu.SemaphoreType.DMA,
            pltpu.SemaphoreType.DMA,
        ],
        compiler_params=pltpu.CompilerParams(needs_layout_passes=False),
    )(_decode_kernel)
    return fn(W_dec, top_idx, top_vals, b_dec)


def _losses(x, sae_out):
    return pl.pallas_call(
        _loss_kernel,
        out_specs=[
            pl.BlockSpec(memory_space=pltpu.SMEM),
            pl.BlockSpec(memory_space=pltpu.SMEM),
        ],
        out_shape=[
            jax.ShapeDtypeStruct((1, 1), jnp.float32),
            jax.ShapeDtypeStruct((1, 1), jnp.float32),
        ],
    )(x, sae_out)


def kernel(x, W_enc, b_enc, W_dec, b_dec):
    top_vals, top_idx = _encode_topk(x, W_enc, b_enc, b_dec)
    sae_out = _decode(W_dec, top_idx, top_vals, b_dec)
    l2, tv = _losses(x, sae_out)
    fvu = (l2 / tv)[0, 0]
    zero = jnp.array(0.0, dtype=sae_out.dtype)
    return (sae_out, top_vals, top_idx, fvu, zero, zero)


# speedup vs baseline: 1.8477x; 1.8477x over previous
import functools

import jax
import jax.numpy as jnp
from jax import lax
from jax.experimental import pallas as pl
from jax.experimental.pallas import tpu as pltpu
from jax.experimental.pallas import tpu_sc as plsc

D_IN = 768
NUM_LATENTS = 12288
K = 32
N_TOKENS = 4096

TOK_BLK = 256
LAT_TILE = 1536

NW = 32
B_PER_W = N_TOKENS // NW
LANES = 16


def _encode_topk_kernel(x_ref, wenc_ref, benc_ref, bdec_ref,
                        vals_ref, idxs_ref, acts_ref):
    B = x_ref.shape[0]
    L = wenc_ref.shape[0]
    sae_in = x_ref[...] - bdec_ref[...]
    for t in range(L // LAT_TILE):
        w = wenc_ref[t * LAT_TILE:(t + 1) * LAT_TILE, :]
        a = lax.dot_general(sae_in, w, (((1,), (1,)), ((), ())),
                            preferred_element_type=jnp.float32)
        b = benc_ref[0, t * LAT_TILE:(t + 1) * LAT_TILE][None, :]
        acts_ref[:, t * LAT_TILE:(t + 1) * LAT_TILE] = jnp.maximum(a + b, 0.0)

    iota = lax.broadcasted_iota(jnp.int32, (B, L), 1)
    kiota = lax.broadcasted_iota(jnp.int32, (B, K), 1)
    big = jnp.int32(2 ** 30)

    def body(k, carry):
        vals, idxs = carry
        a = acts_ref[...]
        m = jnp.max(a, axis=1, keepdims=True)
        idx = jnp.min(jnp.where(a == m, iota, big),
                      axis=1, keepdims=True)
        acts_ref[...] = jnp.where(iota == idx, -jnp.inf, a)
        vals = jnp.where(kiota == k, m, vals)
        idxs = jnp.where(kiota == k, idx, idxs)
        return vals, idxs

    init = (jnp.zeros((B, K), jnp.float32), jnp.zeros((B, K), jnp.int32))
    vals, idxs = lax.fori_loop(0, K, body, init)
    vals_ref[...] = vals
    idxs_ref[...] = idxs


def _loss_kernel(x_ref, out_ref, l2_ref, tv_ref):
    xv = x_ref[...]
    ov = out_ref[...]
    e = ov - xv
    l2_ref[0, 0] = jnp.sum(e * e)
    mu = jnp.mean(xv, axis=0, keepdims=True)
    d = xv - mu
    tv_ref[0, 0] = jnp.sum(d * d)


def _decode_kernel(wdec_hbm, idx_hbm, vals_hbm, bdec_hbm, out_hbm,
                   idx_v, vals_v, rows_v, bcast_v, bdec_v, orow_v, sem0, sem1):
    wid = lax.axis_index("s") * 2 + lax.axis_index("c")
    base = wid * B_PER_W
    pltpu.sync_copy(idx_hbm.at[pl.ds(base, B_PER_W)], idx_v)
    pltpu.sync_copy(vals_hbm.at[pl.ds(base, B_PER_W)], vals_v)
    pltpu.sync_copy(bdec_hbm, bdec_v)

    lane = lax.iota(jnp.int32, LANES)
    sems = (sem0, sem1)

    pltpu.async_copy(wdec_hbm.at[idx_v.at[0]], rows_v.at[0], sems[0])

    def pair_body(p, carry):
        for b in range(2):
            t = 2 * p + b
            nxt = 1 - b

            @pl.when(t + 1 < B_PER_W)
            def _():
                pltpu.async_copy(wdec_hbm.at[idx_v.at[t + 1]],
                                 rows_v.at[nxt], sems[nxt])

            vrow_a = vals_v[t, pl.ds(0, LANES)]
            vrow_b = vals_v[t, pl.ds(LANES, LANES)]
            for k in range(K):
                src = vrow_a if k < LANES else vrow_b
                bval = jnp.sum(jnp.where(lane == (k % LANES), src, 0.0))
                bcast_v[k, :] = jnp.full((LANES,), bval)

            pltpu.make_async_copy(wdec_hbm.at[idx_v.at[t]],
                                  rows_v.at[b], sems[b]).wait()
            rbuf = rows_v.at[b]
            for g in range(3):
                def kbody(k, acc):
                    bval = bcast_v[k, pl.ds(0, LANES)]
                    return tuple(
                        acc[c] + bval * rbuf[k, pl.ds((g * 16 + c) * LANES, LANES)]
                        for c in range(16))
                init = tuple(bdec_v[pl.ds((g * 16 + c) * LANES, LANES)]
                             for c in range(16))
                acc = lax.fori_loop(0, K, kbody, init)
                for c in range(16):
                    orow_v[pl.ds((g * 16 + c) * LANES, LANES)] = acc[c]
            pltpu.sync_copy(orow_v, out_hbm.at[base + t])
        return carry

    lax.fori_loop(0, B_PER_W // 2, pair_body, 0)


def _encode_topk(x, W_enc, b_enc, b_dec):
    grid = (N_TOKENS // TOK_BLK,)
    return pl.pallas_call(
        _encode_topk_kernel,
        grid=grid,
        in_specs=[
            pl.BlockSpec((TOK_BLK, D_IN), lambda i: (i, 0)),
            pl.BlockSpec((NUM_LATENTS, D_IN), lambda i: (0, 0)),
            pl.BlockSpec((1, NUM_LATENTS), lambda i: (0, 0)),
            pl.BlockSpec((1, D_IN), lambda i: (0, 0)),
        ],
        out_specs=[
            pl.BlockSpec((TOK_BLK, K), lambda i: (i, 0)),
            pl.BlockSpec((TOK_BLK, K), lambda i: (i, 0)),
        ],
        out_shape=[
            jax.ShapeDtypeStruct((N_TOKENS, K), jnp.float32),
            jax.ShapeDtypeStruct((N_TOKENS, K), jnp.int32),
        ],
        scratch_shapes=[pltpu.VMEM((TOK_BLK, NUM_LATENTS), jnp.float32)],
        compiler_params=pltpu.CompilerParams(
            vmem_limit_bytes=100 * 1024 * 1024),
    )(x, W_enc, b_enc.reshape(1, NUM_LATENTS), b_dec.reshape(1, D_IN))


def _decode(W_dec, top_idx, top_vals, b_dec):
    mesh = plsc.VectorSubcoreMesh(core_axis_name="c", subcore_axis_name="s")
    fn = functools.partial(
        pl.kernel,
        mesh=mesh,
        out_type=jax.ShapeDtypeStruct((N_TOKENS, D_IN), jnp.float32),
        scratch_types=[
            pltpu.VMEM((B_PER_W, K), jnp.int32),
            pltpu.VMEM((B_PER_W, K), jnp.float32),
            pltpu.VMEM((2, K, D_IN), jnp.float32),
            pltpu.VMEM((K, LANES), jnp.float32),
            pltpu.VMEM((D_IN,), jnp.float32),
            pltpu.VMEM((D_IN,), jnp.float32),
            pltpu.SemaphoreType.DMA,
            pltpu.SemaphoreType.DMA,
        ],
        compiler_params=pltpu.CompilerParams(needs_layout_passes=False),
    )(_decode_kernel)
    return fn(W_dec, top_idx, top_vals, b_dec)


def _losses(x, sae_out):
    return pl.pallas_call(
        _loss_kernel,
        out_specs=[
            pl.BlockSpec(memory_space=pltpu.SMEM),
            pl.BlockSpec(memory_space=pltpu.SMEM),
        ],
        out_shape=[
            jax.ShapeDtypeStruct((1, 1), jnp.float32),
            jax.ShapeDtypeStruct((1, 1), jnp.float32),
        ],
    )(x, sae_out)


def kernel(x, W_enc, b_enc, W_dec, b_dec):
    top_vals, top_idx = _encode_topk(x, W_enc, b_enc, b_dec)
    sae_out = _decode(W_dec, top_idx, top_vals, b_dec)
    l2, tv = _losses(x, sae_out)
    fvu = (l2 / tv)[0, 0]
    zero = jnp.array(0.0, dtype=sae_out.dtype)
    return (sae_out, top_vals, top_idx, fvu, zero, zero)


# speedup vs baseline: 1.8515x; 1.0021x over previous
import functools

import jax
import jax.numpy as jnp
from jax import lax
from jax.experimental import pallas as pl
from jax.experimental.pallas import tpu as pltpu
from jax.experimental.pallas import tpu_sc as plsc

D_IN = 768
NUM_LATENTS = 12288
K = 32
N_TOKENS = 4096

TOK_BLK = 256
LAT_TILE = 1536

NW = 32
B_PER_W = N_TOKENS // NW
LANES = 16


def _encode_topk_kernel(x_ref, wenc_ref, benc_ref, bdec_ref,
                        vals_ref, idxs_ref, acts_ref):
    B = x_ref.shape[0]
    L = wenc_ref.shape[0]
    sae_in = x_ref[...] - bdec_ref[...]
    for t in range(L // LAT_TILE):
        w = wenc_ref[t * LAT_TILE:(t + 1) * LAT_TILE, :]
        a = lax.dot_general(sae_in, w, (((1,), (1,)), ((), ())),
                            preferred_element_type=jnp.float32)
        b = benc_ref[0, t * LAT_TILE:(t + 1) * LAT_TILE][None, :]
        acts_ref[:, t * LAT_TILE:(t + 1) * LAT_TILE] = jnp.maximum(a + b, 0.0)

    iota = lax.broadcasted_iota(jnp.int32, (B, L), 1)
    kiota = lax.broadcasted_iota(jnp.int32, (B, K), 1)
    big = jnp.int32(2 ** 30)

    def body(k, carry):
        vals, idxs = carry
        a = acts_ref[...]
        m = jnp.max(a, axis=1, keepdims=True)
        idx = jnp.min(jnp.where(a == m, iota, big),
                      axis=1, keepdims=True)
        acts_ref[...] = jnp.where(iota == idx, -jnp.inf, a)
        vals = jnp.where(kiota == k, m, vals)
        idxs = jnp.where(kiota == k, idx, idxs)
        return vals, idxs

    init = (jnp.zeros((B, K), jnp.float32), jnp.zeros((B, K), jnp.int32))
    vals, idxs = lax.fori_loop(0, K, body, init)
    vals_ref[...] = vals
    idxs_ref[...] = idxs


def _loss_kernel(x_ref, out_ref, l2_ref, tv_ref):
    xv = x_ref[...]
    ov = out_ref[...]
    e = ov - xv
    l2_ref[0, 0] = jnp.sum(e * e)
    mu = jnp.mean(xv, axis=0, keepdims=True)
    d = xv - mu
    tv_ref[0, 0] = jnp.sum(d * d)


def _decode_kernel(wdec_hbm, idx_hbm, vals_hbm, bdec_hbm, out_hbm,
                   idx_v, vals_v, rows_v, bcast_v, bdec_v, orow0_v, orow1_v,
                   sem0, sem1, osem0, osem1):
    wid = lax.axis_index("s") * 2 + lax.axis_index("c")
    base = wid * B_PER_W
    pltpu.sync_copy(idx_hbm.at[pl.ds(base, B_PER_W)], idx_v)
    pltpu.sync_copy(vals_hbm.at[pl.ds(base, B_PER_W)], vals_v)
    pltpu.sync_copy(bdec_hbm, bdec_v)

    lane = lax.iota(jnp.int32, LANES)
    sems = (sem0, sem1)
    osems = (osem0, osem1)
    obufs = (orow0_v, orow1_v)

    pltpu.async_copy(wdec_hbm.at[idx_v.at[0]], rows_v.at[0], sems[0])

    def pair_body(p, carry):
        for b in range(2):
            t = 2 * p + b
            nxt = 1 - b

            @pl.when(t + 1 < B_PER_W)
            def _():
                pltpu.async_copy(wdec_hbm.at[idx_v.at[t + 1]],
                                 rows_v.at[nxt], sems[nxt])

            vrow_a = vals_v[t, pl.ds(0, LANES)]
            vrow_b = vals_v[t, pl.ds(LANES, LANES)]
            for k in range(K):
                src = vrow_a if k < LANES else vrow_b
                bval = jnp.sum(jnp.where(lane == (k % LANES), src, 0.0))
                bcast_v[k, :] = jnp.full((LANES,), bval)

            pltpu.make_async_copy(wdec_hbm.at[idx_v.at[t]],
                                  rows_v.at[b], sems[b]).wait()

            @pl.when(t >= 2)
            def _():
                pltpu.make_async_copy(obufs[b], out_hbm.at[base + t - 2],
                                      osems[b]).wait()

            rbuf = rows_v.at[b]
            obuf = obufs[b]
            for g in range(3):
                def kbody(k, acc):
                    bval = bcast_v[k, pl.ds(0, LANES)]
                    return tuple(
                        acc[c] + bval * rbuf[k, pl.ds((g * 16 + c) * LANES, LANES)]
                        for c in range(16))
                init = tuple(bdec_v[pl.ds((g * 16 + c) * LANES, LANES)]
                             for c in range(16))
                acc = lax.fori_loop(0, K, kbody, init)
                for c in range(16):
                    obuf[pl.ds((g * 16 + c) * LANES, LANES)] = acc[c]
            pltpu.async_copy(obuf, out_hbm.at[base + t], osems[b])
        return carry

    lax.fori_loop(0, B_PER_W // 2, pair_body, 0)
    for b in range(2):
        pltpu.make_async_copy(obufs[b], out_hbm.at[base + B_PER_W - 2 + b],
                              osems[b]).wait()


def _encode_topk(x, W_enc, b_enc, b_dec):
    grid = (N_TOKENS // TOK_BLK,)
    return pl.pallas_call(
        _encode_topk_kernel,
        grid=grid,
        in_specs=[
            pl.BlockSpec((TOK_BLK, D_IN), lambda i: (i, 0)),
            pl.BlockSpec((NUM_LATENTS, D_IN), lambda i: (0, 0)),
            pl.BlockSpec((1, NUM_LATENTS), lambda i: (0, 0)),
            pl.BlockSpec((1, D_IN), lambda i: (0, 0)),
        ],
        out_specs=[
            pl.BlockSpec((TOK_BLK, K), lambda i: (i, 0)),
            pl.BlockSpec((TOK_BLK, K), lambda i: (i, 0)),
        ],
        out_shape=[
            jax.ShapeDtypeStruct((N_TOKENS, K), jnp.float32),
            jax.ShapeDtypeStruct((N_TOKENS, K), jnp.int32),
        ],
        scratch_shapes=[pltpu.VMEM((TOK_BLK, NUM_LATENTS), jnp.float32)],
        compiler_params=pltpu.CompilerParams(
            vmem_limit_bytes=100 * 1024 * 1024),
    )(x, W_enc, b_enc.reshape(1, NUM_LATENTS), b_dec.reshape(1, D_IN))


def _decode(W_dec, top_idx, top_vals, b_dec):
    mesh = plsc.VectorSubcoreMesh(core_axis_name="c", subcore_axis_name="s")
    fn = functools.partial(
        pl.kernel,
        mesh=mesh,
        out_type=jax.ShapeDtypeStruct((N_TOKENS, D_IN), jnp.float32),
        scratch_types=[
            pltpu.VMEM((B_PER_W, K), jnp.int32),
            pltpu.VMEM((B_PER_W, K), jnp.float32),
            pltpu.VMEM((2, K, D_IN), jnp.float32),
            pltpu.VMEM((K, LANES), jnp.float32),
            pltpu.VMEM((D_IN,), jnp.float32),
            pltpu.VMEM((D_IN,), jnp.float32),
            pltpu.VMEM((D_IN,), jnp.float32),
            pltpu.SemaphoreType.DMA,
            pltpu.SemaphoreType.DMA,
            pltpu.SemaphoreType.DMA,
            pltpu.SemaphoreType.DMA,
        ],
        compiler_params=pltpu.CompilerParams(needs_layout_passes=False),
    )(_decode_kernel)
    return fn(W_dec, top_idx, top_vals, b_dec)


def _losses(x, sae_out):
    return pl.pallas_call(
        _loss_kernel,
        out_specs=[
            pl.BlockSpec(memory_space=pltpu.SMEM),
            pl.BlockSpec(memory_space=pltpu.SMEM),
        ],
        out_shape=[
            jax.ShapeDtypeStruct((1, 1), jnp.float32),
            jax.ShapeDtypeStruct((1, 1), jnp.float32),
        ],
    )(x, sae_out)


def kernel(x, W_enc, b_enc, W_dec, b_dec):
    top_vals, top_idx = _encode_topk(x, W_enc, b_enc, b_dec)
    sae_out = _decode(W_dec, top_idx, top_vals, b_dec)
    l2, tv = _losses(x, sae_out)
    fvu = (l2 / tv)[0, 0]
    zero = jnp.array(0.0, dtype=sae_out.dtype)
    return (sae_out, top_vals, top_idx, fvu, zero, zero)
